# baseline (speedup 1.0000x reference)
import functools

import jax
import jax.numpy as jnp
import numpy as np
from jax import lax
from jax.experimental import pallas as pl
from jax.experimental.pallas import tpu as pltpu
from jax.experimental.pallas import tpu_sc as plsc

N = 10000
D = 128
E = 160000
NUM_CORES = 2
NUM_TILES = 16
CHUNK = 80
NUM_CHUNKS = 25
SUBPHASES = 5
EDGES_PER_TILE = CHUNK * NUM_CHUNKS * SUBPHASES
N_ACC = 10112
ROWS_PER_TILE = N_ACC // NUM_TILES
LAST_ROWS = N - 15 * ROWS_PER_TILE

_ZEROS = np.zeros((N_ACC, D), np.float32)


def _sc_aggregate(node_states, src0, dst0, src1, dst1, zeros):
    mesh = plsc.VectorSubcoreMesh(core_axis_name="c", subcore_axis_name="s")

    @functools.partial(
        pl.kernel,
        mesh=mesh,
        out_type=jax.ShapeDtypeStruct((2 * N, D), jnp.float32),
        scratch_types=[
            pltpu.VMEM_SHARED((N_ACC, D), jnp.float32),
            pltpu.VMEM((NUM_CHUNKS, CHUNK), jnp.int32),
            pltpu.VMEM((NUM_CHUNKS, CHUNK), jnp.int32),
            pltpu.VMEM((CHUNK, D), jnp.float32),
            pltpu.VMEM((CHUNK, D), jnp.float32),
            pltpu.SemaphoreType.DMA,
            pltpu.SemaphoreType.DMA,
        ],
    )
    def agg_kernel(x_hbm, src0_hbm, dst0_hbm, src1_hbm, dst1_hbm, zeros_hbm,
                   out_hbm,
                   acc, src_idx, dst_idx, rows0, rows1, sem0, sem1):
        cid = lax.axis_index("c")
        sid = lax.axis_index("s")
        r0 = sid * ROWS_PER_TILE

        pltpu.sync_copy(zeros_hbm.at[pl.ds(r0, ROWS_PER_TILE)],
                        acc.at[pl.ds(r0, ROWS_PER_TILE)])
        plsc.subcore_barrier()

        bufs = (rows0, rows1)
        sems = (sem0, sem1)

        def start(j, b):
            pltpu.async_copy(x_hbm.at[src_idx.at[j]], bufs[b], sems[b])

        def finish(j, b):
            pltpu.make_async_copy(x_hbm.at[src_idx.at[j]], bufs[b],
                                  sems[b]).wait()
            pltpu.sync_copy(bufs[b], acc.at[dst_idx.at[j]], add=True)

        for p in range(SUBPHASES):
            @pl.when(cid == 0)
            def _():
                pltpu.sync_copy(src0_hbm.at[sid, p], src_idx)
                pltpu.sync_copy(dst0_hbm.at[sid, p], dst_idx)

            @pl.when(cid == 1)
            def _():
                pltpu.sync_copy(src1_hbm.at[sid, p], src_idx)
                pltpu.sync_copy(dst1_hbm.at[sid, p], dst_idx)

            start(0, 0)
            start(1, 1)

            def body(i, carry):
                j = 2 * i
                finish(j, 0)
                start(j + 2, 0)
                finish(j + 1, 1)
                start(j + 3, 1)
                return carry

            lax.fori_loop(0, (NUM_CHUNKS - 3) // 2, body, 0)
            finish(NUM_CHUNKS - 3, 0)
            start(NUM_CHUNKS - 1, 0)
            finish(NUM_CHUNKS - 2, 1)
            finish(NUM_CHUNKS - 1, 0)

        plsc.subcore_barrier()

        @pl.when(sid < NUM_TILES - 1)
        def _():
            pltpu.sync_copy(acc.at[pl.ds(r0, ROWS_PER_TILE)],
                            out_hbm.at[pl.ds(cid * N + r0, ROWS_PER_TILE)])

        @pl.when(sid == NUM_TILES - 1)
        def _():
            pltpu.sync_copy(acc.at[pl.ds(r0, LAST_ROWS)],
                            out_hbm.at[pl.ds(cid * N + r0, LAST_ROWS)])

    return agg_kernel(node_states, src0, dst0, src1, dst1, zeros)


BLOCK_M = 1000
S1_BLOCK_OFF = N // BLOCK_M


def _tc_combine(x, aggs, w_self, w0, w1, b2d):
    def body(x_ref, s0_ref, s1_ref, ws_ref, w0_ref, w1_ref, b_ref, o_ref):
        acc = jnp.dot(x_ref[...], ws_ref[...], preferred_element_type=jnp.float32)
        acc = acc + jnp.dot(s0_ref[...], w0_ref[...], preferred_element_type=jnp.float32)
        acc = acc + jnp.dot(s1_ref[...], w1_ref[...], preferred_element_type=jnp.float32)
        o_ref[...] = jnp.maximum(acc + b_ref[...], 0.0)

    return pl.pallas_call(
        body,
        grid=(N // BLOCK_M,),
        in_specs=[
            pl.BlockSpec((BLOCK_M, D), lambda i: (i, 0)),
            pl.BlockSpec((BLOCK_M, D), lambda i: (i, 0)),
            pl.BlockSpec((BLOCK_M, D), lambda i: (i + S1_BLOCK_OFF, 0)),
            pl.BlockSpec((D, D), lambda i: (0, 0)),
            pl.BlockSpec((D, D), lambda i: (0, 0)),
            pl.BlockSpec((D, D), lambda i: (0, 0)),
            pl.BlockSpec((1, D), lambda i: (0, 0)),
        ],
        out_specs=pl.BlockSpec((BLOCK_M, D), lambda i: (i, 0)),
        out_shape=jax.ShapeDtypeStruct((N, D), jnp.float32),
    )(x, aggs, aggs, w_self, w0, w1, b2d)


def kernel(node_states, adjacency_list_0, adjacency_list_1, node_to_graph_idx,
           W_self, W0, W1, b):
    shape = (NUM_TILES, SUBPHASES, NUM_CHUNKS, CHUNK)
    aggs = _sc_aggregate(
        node_states,
        adjacency_list_0[:, 0].reshape(shape),
        adjacency_list_0[:, 1].reshape(shape),
        adjacency_list_1[:, 0].reshape(shape),
        adjacency_list_1[:, 1].reshape(shape),
        _ZEROS)
    return _tc_combine(node_states, aggs, W_self, W0, W1, b.reshape(1, D))


# speedup vs baseline: 1.1191x; 1.1191x over previous
import functools

import jax
import jax.numpy as jnp
import numpy as np
from jax import lax
from jax.experimental import pallas as pl
from jax.experimental.pallas import tpu as pltpu
from jax.experimental.pallas import tpu_sc as plsc

N = 10000
D = 128
E = 160000
NUM_CORES = 2
NUM_TILES = 16
NUM_WORKERS = NUM_CORES * NUM_TILES
CHUNK = 125
NUM_CHUNKS = 40
SUBPHASES = 2
EDGES_PER_TILE = CHUNK * NUM_CHUNKS * SUBPHASES
N_ACC = 10112
ROWS_PER_TILE = N_ACC // NUM_TILES
LAST_ROWS = N - 15 * ROWS_PER_TILE

_ZEROS = np.zeros((N_ACC, D), np.float32)


def _sc_aggregate(node_states, src_all, dst_all, zeros):
    mesh = plsc.VectorSubcoreMesh(core_axis_name="c", subcore_axis_name="s")

    @functools.partial(
        pl.kernel,
        mesh=mesh,
        out_type=jax.ShapeDtypeStruct((2 * N, D), jnp.float32),
        scratch_types=[
            pltpu.VMEM_SHARED((N_ACC, D), jnp.float32),
            pltpu.VMEM((NUM_CHUNKS, CHUNK), jnp.int32),
            pltpu.VMEM((NUM_CHUNKS, CHUNK), jnp.int32),
            pltpu.VMEM((CHUNK, D), jnp.float32),
            pltpu.VMEM((CHUNK, D), jnp.float32),
            pltpu.SemaphoreType.DMA,
            pltpu.SemaphoreType.DMA,
        ],
    )
    def agg_kernel(x_hbm, src_hbm, dst_hbm, zeros_hbm, out_hbm,
                   acc, src_idx, dst_idx, rows0, rows1, sem0, sem1):
        cid = lax.axis_index("c")
        sid = lax.axis_index("s")
        wid = cid * NUM_TILES + sid
        r0 = sid * ROWS_PER_TILE

        pltpu.sync_copy(zeros_hbm.at[pl.ds(r0, ROWS_PER_TILE)],
                        acc.at[pl.ds(r0, ROWS_PER_TILE)])
        plsc.subcore_barrier()

        bufs = (rows0, rows1)
        sems = (sem0, sem1)

        def start(j, b):
            pltpu.async_copy(x_hbm.at[src_idx.at[j]], bufs[b], sems[b])

        def finish(j, b):
            pltpu.make_async_copy(x_hbm.at[src_idx.at[j]], bufs[b],
                                  sems[b]).wait()
            pltpu.sync_copy(bufs[b], acc.at[dst_idx.at[j]], add=True)

        for p in range(SUBPHASES):
            pltpu.sync_copy(src_hbm.at[SUBPHASES * wid + p], src_idx)
            pltpu.sync_copy(dst_hbm.at[SUBPHASES * wid + p], dst_idx)

            start(0, 0)
            start(1, 1)

            def body(i, carry):
                j = 2 * i
                finish(j, 0)
                start(j + 2, 0)
                finish(j + 1, 1)
                start(j + 3, 1)
                return carry

            lax.fori_loop(0, NUM_CHUNKS // 2 - 1, body, 0)
            finish(NUM_CHUNKS - 2, 0)
            finish(NUM_CHUNKS - 1, 1)

        plsc.subcore_barrier()

        @pl.when(sid < NUM_TILES - 1)
        def _():
            pltpu.sync_copy(acc.at[pl.ds(r0, ROWS_PER_TILE)],
                            out_hbm.at[pl.ds(cid * N + r0, ROWS_PER_TILE)])

        @pl.when(sid == NUM_TILES - 1)
        def _():
            pltpu.sync_copy(acc.at[pl.ds(r0, LAST_ROWS)],
                            out_hbm.at[pl.ds(cid * N + r0, LAST_ROWS)])

    return agg_kernel(node_states, src_all, dst_all, zeros)


BLOCK_M = 1000
S1_BLOCK_OFF = N // BLOCK_M


def _tc_combine(x, aggs, w_self, w0, w1, b2d):
    def body(x_ref, s0_ref, s1_ref, ws_ref, w0_ref, w1_ref, b_ref, o_ref):
        acc = jnp.dot(x_ref[...], ws_ref[...], preferred_element_type=jnp.float32)
        acc = acc + jnp.dot(s0_ref[...], w0_ref[...], preferred_element_type=jnp.float32)
        acc = acc + jnp.dot(s1_ref[...], w1_ref[...], preferred_element_type=jnp.float32)
        o_ref[...] = jnp.maximum(acc + b_ref[...], 0.0)

    return pl.pallas_call(
        body,
        grid=(N // BLOCK_M,),
        in_specs=[
            pl.BlockSpec((BLOCK_M, D), lambda i: (i, 0)),
            pl.BlockSpec((BLOCK_M, D), lambda i: (i, 0)),
            pl.BlockSpec((BLOCK_M, D), lambda i: (i + S1_BLOCK_OFF, 0)),
            pl.BlockSpec((D, D), lambda i: (0, 0)),
            pl.BlockSpec((D, D), lambda i: (0, 0)),
            pl.BlockSpec((D, D), lambda i: (0, 0)),
            pl.BlockSpec((1, D), lambda i: (0, 0)),
        ],
        out_specs=pl.BlockSpec((BLOCK_M, D), lambda i: (i, 0)),
        out_shape=jax.ShapeDtypeStruct((N, D), jnp.float32),
    )(x, aggs, aggs, w_self, w0, w1, b2d)


def kernel(node_states, adjacency_list_0, adjacency_list_1, node_to_graph_idx,
           W_self, W0, W1, b):
    src_all = jnp.concatenate(
        [adjacency_list_0[:, 0], adjacency_list_1[:, 0]]
    ).reshape(SUBPHASES * NUM_WORKERS, NUM_CHUNKS, CHUNK)
    dst_all = jnp.concatenate(
        [adjacency_list_0[:, 1], adjacency_list_1[:, 1]]
    ).reshape(SUBPHASES * NUM_WORKERS, NUM_CHUNKS, CHUNK)
    aggs = _sc_aggregate(node_states, src_all, dst_all, _ZEROS)
    return _tc_combine(node_states, aggs, W_self, W0, W1, b.reshape(1, D))


# speedup vs baseline: 1.1340x; 1.0133x over previous
import functools

import jax
import jax.numpy as jnp
import numpy as np
from jax import lax
from jax.experimental import pallas as pl
from jax.experimental.pallas import tpu as pltpu
from jax.experimental.pallas import tpu_sc as plsc

N = 10000
D = 128
E = 160000
NUM_CORES = 2
NUM_TILES = 16
NUM_WORKERS = NUM_CORES * NUM_TILES
CHUNK = 100
NUM_CHUNKS = 25
SUBPHASES = 4
EDGES_PER_TILE = CHUNK * NUM_CHUNKS * SUBPHASES
N_ACC = 10112
ROWS_PER_TILE = N_ACC // NUM_TILES
LAST_ROWS = N - 15 * ROWS_PER_TILE

_ZEROS = np.zeros((N_ACC, D), np.float32)


def _sc_aggregate(node_states, src_all, dst_all, zeros):
    mesh = plsc.VectorSubcoreMesh(core_axis_name="c", subcore_axis_name="s")

    @functools.partial(
        pl.kernel,
        mesh=mesh,
        out_type=jax.ShapeDtypeStruct((2 * N, D), jnp.float32),
        scratch_types=[
            pltpu.VMEM_SHARED((N_ACC, D), jnp.float32),
            pltpu.VMEM((NUM_CHUNKS, CHUNK), jnp.int32),
            pltpu.VMEM((NUM_CHUNKS, CHUNK), jnp.int32),
            pltpu.VMEM((CHUNK, D), jnp.float32),
            pltpu.VMEM((CHUNK, D), jnp.float32),
            pltpu.VMEM((CHUNK, D), jnp.float32),
            pltpu.SemaphoreType.DMA,
            pltpu.SemaphoreType.DMA,
            pltpu.SemaphoreType.DMA,
        ],
    )
    def agg_kernel(x_hbm, src_hbm, dst_hbm, zeros_hbm, out_hbm,
                   acc, src_idx, dst_idx, rows0, rows1, rows2,
                   sem0, sem1, sem2):
        cid = lax.axis_index("c")
        sid = lax.axis_index("s")
        wid = cid * NUM_TILES + sid
        r0 = sid * ROWS_PER_TILE

        pltpu.sync_copy(zeros_hbm.at[pl.ds(r0, ROWS_PER_TILE)],
                        acc.at[pl.ds(r0, ROWS_PER_TILE)])
        plsc.subcore_barrier()

        bufs = (rows0, rows1, rows2)
        sems = (sem0, sem1, sem2)

        def start(j, b):
            pltpu.async_copy(x_hbm.at[src_idx.at[j]], bufs[b], sems[b])

        def finish(j, b):
            pltpu.make_async_copy(x_hbm.at[src_idx.at[j]], bufs[b],
                                  sems[b]).wait()
            pltpu.sync_copy(bufs[b], acc.at[dst_idx.at[j]], add=True)

        for p in range(SUBPHASES):
            pltpu.sync_copy(src_hbm.at[SUBPHASES * wid + p], src_idx)
            pltpu.sync_copy(dst_hbm.at[SUBPHASES * wid + p], dst_idx)

            start(0, 0)
            start(1, 1)
            start(2, 2)

            def body(i, carry):
                j = 3 * i
                finish(j, 0)
                start(j + 3, 0)
                finish(j + 1, 1)
                start(j + 4, 1)
                finish(j + 2, 2)
                start(j + 5, 2)
                return carry

            iters = (NUM_CHUNKS - 3) // 3
            lax.fori_loop(0, iters, body, 0)
            for j in range(3 * iters, NUM_CHUNKS):
                finish(j, j % 3)
                if j + 3 < NUM_CHUNKS:
                    start(j + 3, (j + 3) % 3)

        plsc.subcore_barrier()

        @pl.when(sid < NUM_TILES - 1)
        def _():
            pltpu.sync_copy(acc.at[pl.ds(r0, ROWS_PER_TILE)],
                            out_hbm.at[pl.ds(cid * N + r0, ROWS_PER_TILE)])

        @pl.when(sid == NUM_TILES - 1)
        def _():
            pltpu.sync_copy(acc.at[pl.ds(r0, LAST_ROWS)],
                            out_hbm.at[pl.ds(cid * N + r0, LAST_ROWS)])

    return agg_kernel(node_states, src_all, dst_all, zeros)


BLOCK_M = 1000
S1_BLOCK_OFF = N // BLOCK_M


def _tc_combine(x, aggs, w_self, w0, w1, b2d):
    def body(x_ref, s0_ref, s1_ref, ws_ref, w0_ref, w1_ref, b_ref, o_ref):
        acc = jnp.dot(x_ref[...], ws_ref[...], preferred_element_type=jnp.float32)
        acc = acc + jnp.dot(s0_ref[...], w0_ref[...], preferred_element_type=jnp.float32)
        acc = acc + jnp.dot(s1_ref[...], w1_ref[...], preferred_element_type=jnp.float32)
        o_ref[...] = jnp.maximum(acc + b_ref[...], 0.0)

    return pl.pallas_call(
        body,
        grid=(N // BLOCK_M,),
        in_specs=[
            pl.BlockSpec((BLOCK_M, D), lambda i: (i, 0)),
            pl.BlockSpec((BLOCK_M, D), lambda i: (i, 0)),
            pl.BlockSpec((BLOCK_M, D), lambda i: (i + S1_BLOCK_OFF, 0)),
            pl.BlockSpec((D, D), lambda i: (0, 0)),
            pl.BlockSpec((D, D), lambda i: (0, 0)),
            pl.BlockSpec((D, D), lambda i: (0, 0)),
            pl.BlockSpec((1, D), lambda i: (0, 0)),
        ],
        out_specs=pl.BlockSpec((BLOCK_M, D), lambda i: (i, 0)),
        out_shape=jax.ShapeDtypeStruct((N, D), jnp.float32),
    )(x, aggs, aggs, w_self, w0, w1, b2d)


def kernel(node_states, adjacency_list_0, adjacency_list_1, node_to_graph_idx,
           W_self, W0, W1, b):
    src_all = jnp.concatenate(
        [adjacency_list_0[:, 0], adjacency_list_1[:, 0]]
    ).reshape(SUBPHASES * NUM_WORKERS, NUM_CHUNKS, CHUNK)
    dst_all = jnp.concatenate(
        [adjacency_list_0[:, 1], adjacency_list_1[:, 1]]
    ).reshape(SUBPHASES * NUM_WORKERS, NUM_CHUNKS, CHUNK)
    aggs = _sc_aggregate(node_states, src_all, dst_all, _ZEROS)
    return _tc_combine(node_states, aggs, W_self, W0, W1, b.reshape(1, D))


# speedup vs baseline: 1.1522x; 1.0160x over previous
import functools

import jax
import jax.numpy as jnp
import numpy as np
from jax import lax
from jax.experimental import pallas as pl
from jax.experimental.pallas import tpu as pltpu
from jax.experimental.pallas import tpu_sc as plsc

N = 10000
D = 128
E = 160000
NUM_CORES = 2
NUM_TILES = 16
NUM_WORKERS = NUM_CORES * NUM_TILES
CHUNK = 100
NUM_CHUNKS = 25
SUBPHASES = 4
EDGES_PER_TILE = CHUNK * NUM_CHUNKS * SUBPHASES
N_ACC = 10112
ROWS_PER_TILE = N_ACC // NUM_TILES
LAST_ROWS = N - 15 * ROWS_PER_TILE

_ZEROS = np.zeros((N_ACC, D), np.float32)


def _sc_aggregate(node_states, src_all, dst_all, zeros):
    mesh = plsc.VectorSubcoreMesh(core_axis_name="c", subcore_axis_name="s")

    @functools.partial(
        pl.kernel,
        mesh=mesh,
        out_type=jax.ShapeDtypeStruct((2 * N, D), jnp.float32),
        scratch_types=[
            pltpu.VMEM_SHARED((N_ACC, D), jnp.float32),
            pltpu.VMEM((NUM_CHUNKS, CHUNK), jnp.int32),
            pltpu.VMEM((NUM_CHUNKS, CHUNK), jnp.int32),
            pltpu.VMEM((CHUNK, D), jnp.float32),
            pltpu.VMEM((CHUNK, D), jnp.float32),
            pltpu.VMEM((CHUNK, D), jnp.float32),
            pltpu.SemaphoreType.DMA,
            pltpu.SemaphoreType.DMA,
            pltpu.SemaphoreType.DMA,
        ],
    )
    def agg_kernel(x_hbm, src_hbm, dst_hbm, zeros_hbm, out_hbm,
                   acc, src_idx, dst_idx, rows0, rows1, rows2,
                   sem0, sem1, sem2):
        cid = lax.axis_index("c")
        sid = lax.axis_index("s")
        wid = cid * NUM_TILES + sid
        r0 = sid * ROWS_PER_TILE

        pltpu.sync_copy(zeros_hbm.at[pl.ds(r0, ROWS_PER_TILE)],
                        acc.at[pl.ds(r0, ROWS_PER_TILE)])
        plsc.subcore_barrier()

        bufs = (rows0, rows1, rows2)
        sems = (sem0, sem1, sem2)

        def start(j, b):
            pltpu.async_copy(x_hbm.at[src_idx.at[j]], bufs[b], sems[b])

        def finish(j, b):
            pltpu.make_async_copy(x_hbm.at[src_idx.at[j]], bufs[b],
                                  sems[b]).wait()
            pltpu.sync_copy(bufs[b], acc.at[dst_idx.at[j]], add=True)

        for p in range(SUBPHASES):
            pltpu.sync_copy(src_hbm.at[SUBPHASES * wid + p], src_idx)
            pltpu.sync_copy(dst_hbm.at[SUBPHASES * wid + p], dst_idx)

            start(0, 0)
            start(1, 1)
            start(2, 2)

            def body(i, carry):
                j = 3 * i
                finish(j, 0)
                start(j + 3, 0)
                finish(j + 1, 1)
                start(j + 4, 1)
                finish(j + 2, 2)
                start(j + 5, 2)
                return carry

            iters = (NUM_CHUNKS - 3) // 3
            lax.fori_loop(0, iters, body, 0)
            for j in range(3 * iters, NUM_CHUNKS):
                finish(j, j % 3)
                if j + 3 < NUM_CHUNKS:
                    start(j + 3, (j + 3) % 3)

        plsc.subcore_barrier()

        @pl.when(sid < NUM_TILES - 1)
        def _():
            pltpu.sync_copy(acc.at[pl.ds(r0, ROWS_PER_TILE)],
                            out_hbm.at[pl.ds(cid * N + r0, ROWS_PER_TILE)])

        @pl.when(sid == NUM_TILES - 1)
        def _():
            pltpu.sync_copy(acc.at[pl.ds(r0, LAST_ROWS)],
                            out_hbm.at[pl.ds(cid * N + r0, LAST_ROWS)])

    return agg_kernel(node_states, src_all, dst_all, zeros)


BLOCK_M = 1000
S1_BLOCK_OFF = N // BLOCK_M


def _tc_self(x, w_self, b2d):
    def body(x_ref, ws_ref, b_ref, y_ref):
        y_ref[...] = (
            jnp.dot(x_ref[...], ws_ref[...], preferred_element_type=jnp.float32)
            + b_ref[...])

    return pl.pallas_call(
        body,
        grid=(N // BLOCK_M,),
        in_specs=[
            pl.BlockSpec((BLOCK_M, D), lambda i: (i, 0)),
            pl.BlockSpec((D, D), lambda i: (0, 0)),
            pl.BlockSpec((1, D), lambda i: (0, 0)),
        ],
        out_specs=pl.BlockSpec((BLOCK_M, D), lambda i: (i, 0)),
        out_shape=jax.ShapeDtypeStruct((N, D), jnp.float32),
    )(x, w_self, b2d)


def _tc_combine(y, aggs, w0, w1):
    def body(y_ref, s0_ref, s1_ref, w0_ref, w1_ref, o_ref):
        acc = y_ref[...]
        acc = acc + jnp.dot(s0_ref[...], w0_ref[...], preferred_element_type=jnp.float32)
        acc = acc + jnp.dot(s1_ref[...], w1_ref[...], preferred_element_type=jnp.float32)
        o_ref[...] = jnp.maximum(acc, 0.0)

    return pl.pallas_call(
        body,
        grid=(N // BLOCK_M,),
        in_specs=[
            pl.BlockSpec((BLOCK_M, D), lambda i: (i, 0)),
            pl.BlockSpec((BLOCK_M, D), lambda i: (i, 0)),
            pl.BlockSpec((BLOCK_M, D), lambda i: (i + S1_BLOCK_OFF, 0)),
            pl.BlockSpec((D, D), lambda i: (0, 0)),
            pl.BlockSpec((D, D), lambda i: (0, 0)),
        ],
        out_specs=pl.BlockSpec((BLOCK_M, D), lambda i: (i, 0)),
        out_shape=jax.ShapeDtypeStruct((N, D), jnp.float32),
    )(y, aggs, aggs, w0, w1)


def kernel(node_states, adjacency_list_0, adjacency_list_1, node_to_graph_idx,
           W_self, W0, W1, b):
    src_all = jnp.concatenate(
        [adjacency_list_0[:, 0], adjacency_list_1[:, 0]]
    ).reshape(SUBPHASES * NUM_WORKERS, NUM_CHUNKS, CHUNK)
    dst_all = jnp.concatenate(
        [adjacency_list_0[:, 1], adjacency_list_1[:, 1]]
    ).reshape(SUBPHASES * NUM_WORKERS, NUM_CHUNKS, CHUNK)
    y = _tc_self(node_states, W_self, b.reshape(1, D))
    aggs = _sc_aggregate(node_states, src_all, dst_all, _ZEROS)
    return _tc_combine(y, aggs, W0, W1)
